# baseline (speedup 1.0000x reference)
import functools

import jax
import jax.numpy as jnp
from jax import lax
from jax.experimental import pallas as pl
from jax.experimental.pallas import tpu as pltpu
from jax.experimental.pallas import tpu_sc as plsc

_NC = 2
_NS = 16
_NW = _NC * _NS
_NI = 1000
_NP = 16
_NT = 4096
_TT = _NT // 128


def _sc_body(x5, c2, out3, buf, cvec, outbuf):
    wid = lax.axis_index("s") * _NC + lax.axis_index("c")

    def per_item(k, carry):
        i = wid + _NW * k

        @pl.when(i < _NI)
        def _():
            pltpu.sync_copy(x5.at[i], buf)
            pltpu.sync_copy(c2.at[pl.ds(2 * i, 2)], cvec)
            cps = [cvec[h, pl.ds(16 * g, 16)]
                   for h in range(2) for g in range(8)]

            def per_tt(tt, carry2):
                for v2 in range(8):
                    acc = jnp.zeros((16,), jnp.float32)
                    for ph in range(2):
                        for pi in range(8):
                            acc = (acc + buf[ph, tt, pi, pl.ds(16 * v2, 16)]
                                   * cps[ph * 8 + pi])
                    outbuf[tt, pl.ds(16 * v2, 16)] = acc
                return carry2

            lax.fori_loop(0, _TT, per_tt, 0)
            pltpu.sync_copy(outbuf, out3.at[i])

        return carry

    lax.fori_loop(0, (_NI + _NW - 1) // _NW, per_item, 0)


def kernel(x, coef):
    num_trips, num_items, num_params = x.shape
    xt = jnp.transpose(x, (1, 2, 0))
    x5 = xt.reshape(num_items, 2, 8, _TT, 128).transpose(0, 1, 3, 2, 4)
    c2 = jnp.repeat(coef.reshape(num_items, num_params), 16,
                    axis=-1).reshape(2 * num_items, 128)

    mesh = plsc.VectorSubcoreMesh(core_axis_name="c", subcore_axis_name="s")
    run = functools.partial(
        pl.kernel,
        mesh=mesh,
        out_type=jax.ShapeDtypeStruct((num_items, _TT, 128), jnp.float32),
        scratch_types=[
            pltpu.VMEM((2, _TT, 8, 128), jnp.float32),
            pltpu.VMEM((2, 128), jnp.float32),
            pltpu.VMEM((_TT, 128), jnp.float32),
        ],
    )(_sc_body)
    out3 = run(x5, c2)
    return out3.reshape(num_items, num_trips).T


# speedup vs baseline: 4.0947x; 4.0947x over previous
import jax
import jax.numpy as jnp
from jax.experimental import pallas as pl

_IB = 64


def _body(x_ref, c_ref, o_ref):
    o_ref[...] = jnp.sum(x_ref[...] * c_ref[...][:, :, None], axis=1)


def kernel(x, coef):
    num_trips, num_items, num_params = x.shape
    xt = jnp.transpose(x, (1, 2, 0))
    tbc = num_trips // 2
    outT = pl.pallas_call(
        _body,
        grid=(pl.cdiv(num_items, _IB), 2),
        in_specs=[
            pl.BlockSpec((_IB, num_params, tbc), lambda i, j: (i, 0, j)),
            pl.BlockSpec((_IB, num_params), lambda i, j: (i, 0)),
        ],
        out_specs=pl.BlockSpec((_IB, tbc), lambda i, j: (i, j)),
        out_shape=jax.ShapeDtypeStruct((num_items, num_trips), jnp.float32),
    )(xt, coef)
    return outT.T
